# baseline (speedup 1.0000x reference)
import functools

import jax
import jax.numpy as jnp
from jax import lax
from jax.experimental import pallas as pl
from jax.experimental.pallas import tpu as pltpu
from jax.experimental.pallas import tpu_sc as plsc

NUM_SAMPLES = 16
LANES = 16
NUM_CORES = 2
NUM_SUBCORES = 16
NW = NUM_CORES * NUM_SUBCORES
GATHER_CHUNK = 128


@functools.partial(jax.jit, static_argnames=("batch", "degree"))
def _sample_sc(adj_info, node_ids, cols, *, batch, degree):
    bpw = batch // NW

    mesh = plsc.VectorSubcoreMesh(core_axis_name="c", subcore_axis_name="s")

    @functools.partial(
        pl.kernel,
        mesh=mesh,
        compiler_params=pltpu.CompilerParams(use_tc_tiling_on_sc=False),
        out_type=jax.ShapeDtypeStruct((batch, NUM_SAMPLES), jnp.int32),
        scratch_types=[
            pltpu.VMEM((bpw,), jnp.int32),
            pltpu.VMEM((bpw, degree), jnp.int32),
            pltpu.VMEM((bpw, NUM_SAMPLES), jnp.int32),
            pltpu.VMEM((LANES,), jnp.int32),
            pltpu.SemaphoreType.DMA,
        ],
    )
    def body(adj_hbm, nid_hbm, cols_hbm, out_hbm, nid_v, rows_v, out_v, cols_v, sem):
        wid = lax.axis_index("s") * NUM_CORES + lax.axis_index("c")
        base = wid * bpw
        pltpu.sync_copy(nid_hbm.at[pl.ds(base, bpw)], nid_v)
        pltpu.sync_copy(cols_hbm, cols_v)

        copies = []
        for c in range(0, bpw, GATHER_CHUNK):
            copies.append(
                pltpu.async_copy(
                    adj_hbm.at[nid_v.at[pl.ds(c, GATHER_CHUNK)]],
                    rows_v.at[pl.ds(c, GATHER_CHUNK)],
                    sem,
                )
            )
        for cp in copies:
            cp.wait()

        cols_vec = cols_v[...]
        in_lo = cols_vec < LANES
        lo_idx = jnp.where(in_lo, cols_vec, 0)
        hi_idx = jnp.where(in_lo, 0, cols_vec - LANES)
        dnums = lax.GatherDimensionNumbers(
            offset_dims=(), collapsed_slice_dims=(0,), start_index_map=(0,)
        )

        def vgather(vec, idx):
            return lax.gather(
                vec, idx[:, None], dnums, slice_sizes=(1,),
                mode=lax.GatherScatterMode.PROMISE_IN_BOUNDS,
            )

        def step(i, carry):
            lo = rows_v[i, pl.ds(0, LANES)]
            hi = rows_v[i, pl.ds(LANES, LANES)]
            out_v[i] = jnp.where(in_lo, vgather(lo, lo_idx), vgather(hi, hi_idx))
            return carry

        lax.fori_loop(0, bpw, step, 0)
        pltpu.sync_copy(out_v, out_hbm.at[pl.ds(base, bpw)])

    return body(adj_info, node_ids, cols)


def kernel(adj_info, node_ids, num_samples):
    batch = node_ids.shape[0]
    degree = adj_info.shape[1]
    perm = jax.random.permutation(jax.random.key(42), degree).astype(jnp.int32)
    start = jnp.asarray(num_samples, jnp.int32) - NUM_SAMPLES
    cols = lax.dynamic_slice_in_dim(perm, start, NUM_SAMPLES)
    return _sample_sc(
        adj_info.astype(jnp.int32),
        node_ids.astype(jnp.int32),
        cols,
        batch=batch,
        degree=degree,
    )
